# Initial kernel scaffold; baseline (speedup 1.0000x reference)
import functools

import jax
import jax.numpy as jnp
from jax import lax
from jax.experimental import pallas as pl
from jax.experimental.pallas import tpu as pltpu
from jax.experimental.pallas import tpu_sc as plsc

B = 4
T = 8
N = 2048
K = 8
M = B * N
HALF = T // 2

NC = 2
NS = 16
NW = NC * NS
PTS_PER_W = M // NW
CHUNK = 16

BN = 256
BM = 512

_f32 = jnp.float32
_HIGH = lax.Precision.HIGHEST


def _knn_body(p1_ref, p2t_ref, idx_ref):
    b = pl.program_id(0)
    p1 = p1_ref[0]
    p2t = p2t_ref[0]
    sq2 = jnp.sum(p2t * p2t, axis=0)
    dots = jnp.dot(p1, p2t, precision=_HIGH)
    scores = sq2[None, :] - 2.0 * dots
    cols = lax.broadcasted_iota(jnp.int32, (BN, N), 1)
    base = b * N
    for j in range(K):
        m = jnp.min(scores, axis=1, keepdims=True)
        cand = jnp.where(scores == m, cols, N)
        sel = jnp.min(cand, axis=1)
        idx_ref[0, j, :] = sel + base
        scores = jnp.where(cols == sel[:, None], jnp.inf, scores)


def _knn(p1n, p2t):
    return pl.pallas_call(
        _knn_body,
        grid=(B, N // BN),
        in_specs=[
            pl.BlockSpec((1, BN, 8), lambda b, nb: (b, nb, 0)),
            pl.BlockSpec((1, 8, N), lambda b, nb: (b, 0, 0)),
        ],
        out_specs=pl.BlockSpec((1, K, BN), lambda b, nb: (b, 0, nb)),
        out_shape=jax.ShapeDtypeStruct((B, K, N), jnp.int32),
    )(p1n, p2t)


def _prep_body(s1_ref, s2_ref, s3_ref, p2_ref,
               w1s_ref, w1p_ref, w2s_ref, w2p_ref, w3s_ref, w3p_ref,
               a1_ref, a2_ref, a3_ref):
    p2 = p2_ref[...]
    a1_ref[...] = (jnp.dot(s1_ref[...], w1s_ref[...], precision=_HIGH)
                   + jnp.dot(p2, w1p_ref[...], precision=_HIGH))
    a2_ref[...] = (jnp.dot(s2_ref[...], w2s_ref[...], precision=_HIGH)
                   + jnp.dot(p2, w2p_ref[...], precision=_HIGH))
    a3_ref[...] = (jnp.dot(s3_ref[...], w3s_ref[...], precision=_HIGH)
                   + jnp.dot(p2, w3p_ref[...], precision=_HIGH))


def _full(r, c):
    return pl.BlockSpec((r, c), lambda i: (0, 0))


def _prep(s1, s2, s3, p2f, w1s, w1p, w2s, w2p, w3s, w3p):
    return pl.pallas_call(
        _prep_body,
        grid=(M // BM,),
        in_specs=[
            pl.BlockSpec((BM, 64), lambda i: (i, 0)),
            pl.BlockSpec((BM, 128), lambda i: (i, 0)),
            pl.BlockSpec((BM, 256), lambda i: (i, 0)),
            pl.BlockSpec((BM, 8), lambda i: (i, 0)),
            _full(64, 64), _full(8, 64),
            _full(128, 128), _full(8, 128),
            _full(256, 256), _full(8, 256),
        ],
        out_specs=[
            pl.BlockSpec((BM, 64), lambda i: (i, 0)),
            pl.BlockSpec((BM, 128), lambda i: (i, 0)),
            pl.BlockSpec((BM, 256), lambda i: (i, 0)),
        ],
        out_shape=[
            jax.ShapeDtypeStruct((M, 64), _f32),
            jax.ShapeDtypeStruct((M, 128), _f32),
            jax.ShapeDtypeStruct((M, 256), _f32),
        ],
    )(s1, s2, s3, p2f, w1s, w1p, w2s, w2p, w3s, w3p)


_sc_mesh = plsc.VectorSubcoreMesh(
    core_axis_name="c", subcore_axis_name="s", num_cores=NC, num_subcores=NS)


@functools.partial(
    pl.kernel,
    out_type=[
        jax.ShapeDtypeStruct((M, 64), _f32),
        jax.ShapeDtypeStruct((M, 128), _f32),
        jax.ShapeDtypeStruct((M, 256), _f32),
    ],
    mesh=_sc_mesh,
    scratch_types=[
        pltpu.VMEM((CHUNK * K,), jnp.int32),
        pltpu.VMEM((CHUNK * K, 64), _f32),
        pltpu.VMEM((CHUNK * K, 128), _f32),
        pltpu.VMEM((CHUNK * K, 256), _f32),
        pltpu.VMEM((CHUNK, 64), _f32),
        pltpu.VMEM((CHUNK, 128), _f32),
        pltpu.VMEM((CHUNK, 256), _f32),
        pltpu.SemaphoreType.DMA,
        pltpu.SemaphoreType.DMA,
        pltpu.SemaphoreType.DMA,
    ],
)
def _gather_max(a1_hbm, a2_hbm, a3_hbm, idx_hbm,
                g1_hbm, g2_hbm, g3_hbm,
                idx_v, r1, r2, r3, o1, o2, o3, sem1, sem2, sem3):
    wid = lax.axis_index("s") * NC + lax.axis_index("c")
    base_pt = wid * PTS_PER_W

    def chunk_body(ci, carry):
        pt0 = base_pt + ci * CHUNK
        pltpu.sync_copy(idx_hbm.at[pl.ds(pt0 * K, CHUNK * K)], idx_v)
        cp1 = pltpu.async_copy(a1_hbm.at[idx_v], r1, sem1)
        cp2 = pltpu.async_copy(a2_hbm.at[idx_v], r2, sem2)
        cp3 = pltpu.async_copy(a3_hbm.at[idx_v], r3, sem3)
        cp1.wait()
        cp2.wait()
        cp3.wait()

        def pt_body(p, c2):
            row = p * K
            for r, o, cdim in ((r1, o1, 64), (r2, o2, 128), (r3, o3, 256)):
                for ch in range(cdim // 16):
                    sl = pl.ds(ch * 16, 16)
                    v = r[row, sl]
                    for j in range(1, K):
                        v = jnp.maximum(v, r[row + j, sl])
                    o[p, sl] = v
            return c2

        lax.fori_loop(0, CHUNK, pt_body, 0)
        pltpu.sync_copy(o1, g1_hbm.at[pl.ds(pt0, CHUNK)])
        pltpu.sync_copy(o2, g2_hbm.at[pl.ds(pt0, CHUNK)])
        pltpu.sync_copy(o3, g3_hbm.at[pl.ds(pt0, CHUNK)])
        return carry

    lax.fori_loop(0, PTS_PER_W // CHUNK, chunk_body, 0)


def _post_body(decode, g1_ref, g2_ref, g3_ref, p1_ref,
               w1p_ref, w2x_ref, w2p_ref, w3x_ref, w3p_ref,
               b1_ref, b2_ref, b3_ref,
               wm_ref, bm_ref, wl_ref, bl_ref,
               s1_ref, s2_ref, s3_ref, np_ref=None):
    p1 = p1_ref[...]
    s1 = g1_ref[...] + b1_ref[...] - jnp.dot(p1, w1p_ref[...], precision=_HIGH)
    s1_ref[...] = s1
    s2 = (g2_ref[...] + b2_ref[...]
          + jnp.dot(s1, w2x_ref[...], precision=_HIGH)
          - jnp.dot(p1, w2p_ref[...], precision=_HIGH))
    s2_ref[...] = s2
    s3 = (g3_ref[...] + b3_ref[...]
          + jnp.dot(s2, w3x_ref[...], precision=_HIGH)
          - jnp.dot(p1, w3p_ref[...], precision=_HIGH))
    s3_ref[...] = s3
    if decode:
        h = jnp.maximum(jnp.dot(s3, wm_ref[...], precision=_HIGH)
                        + bm_ref[...], 0.0)
        motion = jnp.dot(h, wl_ref[...], precision=_HIGH) + bl_ref[...]
        np_ref[...] = p1 + motion


def _post(decode, g1, g2, g3, p1f, w1p, w2x, w2p, w3x, w3p,
          b1r, b2r, b3r, wm, bmr, wlp, blr):
    out_specs = [
        pl.BlockSpec((BM, 64), lambda i: (i, 0)),
        pl.BlockSpec((BM, 128), lambda i: (i, 0)),
        pl.BlockSpec((BM, 256), lambda i: (i, 0)),
    ]
    out_shape = [
        jax.ShapeDtypeStruct((M, 64), _f32),
        jax.ShapeDtypeStruct((M, 128), _f32),
        jax.ShapeDtypeStruct((M, 256), _f32),
    ]
    if decode:
        out_specs.append(pl.BlockSpec((BM, 8), lambda i: (i, 0)))
        out_shape.append(jax.ShapeDtypeStruct((M, 8), _f32))
    return pl.pallas_call(
        functools.partial(_post_body, decode),
        grid=(M // BM,),
        in_specs=[
            pl.BlockSpec((BM, 64), lambda i: (i, 0)),
            pl.BlockSpec((BM, 128), lambda i: (i, 0)),
            pl.BlockSpec((BM, 256), lambda i: (i, 0)),
            pl.BlockSpec((BM, 8), lambda i: (i, 0)),
            _full(8, 64), _full(64, 128), _full(8, 128),
            _full(128, 256), _full(8, 256),
            _full(1, 64), _full(1, 128), _full(1, 256),
            _full(256, 64), _full(1, 64), _full(64, 8), _full(1, 8),
        ],
        out_specs=out_specs,
        out_shape=out_shape,
    )(g1, g2, g3, p1f, w1p, w2x, w2p, w3x, w3p,
      b1r, b2r, b3r, wm, bmr, wlp, blr)


def kernel(frames, W1, b1, W2, b2, W3, b3, Wm, bm, Wl, bl):
    pad35 = ((0, 5), (0, 0))
    w1s, w1p = W1[:64], jnp.pad(W1[64:], pad35)
    w2s, w2x, w2p = W2[:128], W2[128:192], jnp.pad(W2[192:], pad35)
    w3s, w3x, w3p = W3[:256], W3[256:384], jnp.pad(W3[384:], pad35)
    wlp = jnp.pad(Wl, ((0, 0), (0, 5)))
    blr = jnp.pad(bl, (0, 5))[None, :]
    b1r, b2r, b3r, bmr = b1[None, :], b2[None, :], b3[None, :], bm[None, :]

    framesP = jnp.pad(frames, ((0, 0), (0, 0), (0, 0), (0, 5)))
    framesT = framesP.transpose(0, 1, 3, 2)

    s1 = jnp.zeros((M, 64), _f32)
    s2 = jnp.zeros((M, 128), _f32)
    s3 = jnp.zeros((M, 256), _f32)

    preds = []
    p1n, p1t = framesP[:, 0], framesT[:, 0]
    p2t_t = p1t
    p1f = p1n.reshape(M, 8)
    p2f = p1f

    for t in range(T):
        idx_t = _knn(p1n, p2t_t)
        idx_flat = idx_t.transpose(0, 2, 1).reshape(M * K)
        a1, a2, a3 = _prep(s1, s2, s3, p2f,
                           w1s, w1p, w2s, w2p, w3s, w3p)
        g1, g2, g3 = _gather_max(a1, a2, a3, idx_flat)
        decode = t >= HALF
        outs = _post(decode, g1, g2, g3, p1f,
                     w1p, w2x, w2p, w3x, w3p,
                     b1r, b2r, b3r, Wm, bmr, wlp, blr)
        if decode:
            s1, s2, s3, newpf = outs
            preds.append(newpf[:, :3].reshape(B, N, 3))
        else:
            s1, s2, s3 = outs

        p2f = p1f
        p2t_t = p1t
        if t + 1 < HALF:
            p1n, p1t = framesP[:, t + 1], framesT[:, t + 1]
            p1f = p1n.reshape(M, 8)
        elif t + 1 == HALF:
            p1n, p1t = framesP[:, HALF - 1], framesT[:, HALF - 1]
            p1f = p1n.reshape(M, 8)
            p2t_t = p1t
            p2f = p1f
        else:
            p1f = newpf
            p1n = newpf.reshape(B, N, 8)
            p1t = p1n.transpose(0, 2, 1)

    return jnp.stack(preds, axis=1)


# speedup vs baseline: 16.8237x; 16.8237x over previous
import functools

import jax
import jax.numpy as jnp
from jax import lax
from jax.experimental import pallas as pl
from jax.experimental.pallas import tpu as pltpu
from jax.experimental.pallas import tpu_sc as plsc

B = 4
T = 8
N = 2048
K = 8
M = B * N
HALF = T // 2

NC = 2
NS = 16
NW = NC * NS
PTS_PER_W = M // NW
CHUNK = 16

BN = 256
BM = 512

_f32 = jnp.float32
_HIGH = lax.Precision.HIGHEST


def _knn_body(p1_ref, p2t_ref, idx_ref):
    b = pl.program_id(0)
    p1 = p1_ref[0]
    p2t = p2t_ref[0]
    sq2 = jnp.sum(p2t * p2t, axis=0)
    dots = jnp.dot(p1, p2t, precision=_HIGH)
    scores = sq2[None, :] - 2.0 * dots
    cols = lax.broadcasted_iota(jnp.int32, (BN, N), 1)
    base = b * N
    for j in range(K):
        m = jnp.min(scores, axis=1, keepdims=True)
        cand = jnp.where(scores == m, cols, N)
        sel = jnp.min(cand, axis=1)
        idx_ref[0, j, :] = sel + base
        scores = jnp.where(cols == sel[:, None], jnp.inf, scores)


def _knn(p1n, p2t):
    return pl.pallas_call(
        _knn_body,
        grid=(B, N // BN),
        in_specs=[
            pl.BlockSpec((1, BN, 8), lambda b, nb: (b, nb, 0)),
            pl.BlockSpec((1, 8, N), lambda b, nb: (b, 0, 0)),
        ],
        out_specs=pl.BlockSpec((1, K, BN), lambda b, nb: (b, 0, nb)),
        out_shape=jax.ShapeDtypeStruct((B, K, N), jnp.int32),
    )(p1n, p2t)


CT = 512


def _prep_body(s1_ref, s2_ref, s3_ref, p2_ref,
               w1s_ref, w1p_ref, w2s_ref, w2p_ref, w3s_ref, w3p_ref,
               at_ref):
    p2 = p2_ref[...]
    at_ref[:, 0:128] = (jnp.dot(s1_ref[...], w1s_ref[...], precision=_HIGH)
                        + jnp.dot(p2, w1p_ref[...], precision=_HIGH))
    at_ref[:, 128:256] = (jnp.dot(s2_ref[...], w2s_ref[...], precision=_HIGH)
                          + jnp.dot(p2, w2p_ref[...], precision=_HIGH))
    at_ref[:, 256:512] = (jnp.dot(s3_ref[...], w3s_ref[...], precision=_HIGH)
                          + jnp.dot(p2, w3p_ref[...], precision=_HIGH))


def _full(r, c):
    return pl.BlockSpec((r, c), lambda i: (0, 0))


def _prep(s1, s2, s3, p2f, w1s, w1p, w2s, w2p, w3s, w3p):
    return pl.pallas_call(
        _prep_body,
        grid=(M // BM,),
        in_specs=[
            pl.BlockSpec((BM, 64), lambda i: (i, 0)),
            pl.BlockSpec((BM, 128), lambda i: (i, 0)),
            pl.BlockSpec((BM, 256), lambda i: (i, 0)),
            pl.BlockSpec((BM, 8), lambda i: (i, 0)),
            _full(64, 128), _full(8, 128),
            _full(128, 128), _full(8, 128),
            _full(256, 256), _full(8, 256),
        ],
        out_specs=pl.BlockSpec((BM, CT), lambda i: (i, 0)),
        out_shape=jax.ShapeDtypeStruct((M, CT), _f32),
    )(s1, s2, s3, p2f, w1s, w1p, w2s, w2p, w3s, w3p)


_LIVE_CH = tuple(range(4)) + tuple(range(8, CT // 16))


@functools.cache
def _gather_max_kernel():
    mesh = plsc.VectorSubcoreMesh(
        core_axis_name="c", subcore_axis_name="s",
        num_cores=NC, num_subcores=NS)

    @functools.partial(
        pl.kernel,
        out_type=jax.ShapeDtypeStruct((M, CT), _f32),
        mesh=mesh,
        scratch_types=[
            pltpu.VMEM((CHUNK * K,), jnp.int32),
            pltpu.VMEM((CHUNK * K, CT), _f32),
            pltpu.VMEM((CHUNK, CT), _f32),
            pltpu.SemaphoreType.DMA,
        ],
    )
    def _gather_max(at_hbm, idx_hbm, gt_hbm, idx_v, rows_v, out_v, sem):
        wid = lax.axis_index("s") * NC + lax.axis_index("c")
        base_pt = wid * PTS_PER_W

        def chunk_body(ci, carry):
            pt0 = base_pt + ci * CHUNK
            pltpu.sync_copy(idx_hbm.at[pl.ds(pt0 * K, CHUNK * K)], idx_v)
            pltpu.async_copy(at_hbm.at[idx_v], rows_v, sem).wait()

            def pt_body(p, c2):
                row = p * K
                for ch in _LIVE_CH:
                    sl = pl.ds(ch * 16, 16)
                    v = rows_v[row, sl]
                    for j in range(1, K):
                        v = jnp.maximum(v, rows_v[row + j, sl])
                    out_v[p, sl] = v
                return c2

            lax.fori_loop(0, CHUNK, pt_body, 0)
            pltpu.sync_copy(out_v, gt_hbm.at[pl.ds(pt0, CHUNK)])
            return carry

        lax.fori_loop(0, PTS_PER_W // CHUNK, chunk_body, 0)

    return _gather_max


def _post_body(decode, gt_ref, p1_ref,
               w1p_ref, w2x_ref, w2p_ref, w3x_ref, w3p_ref,
               b1_ref, b2_ref, b3_ref,
               wm_ref, bm_ref, wl_ref, bl_ref,
               s1_ref, s2_ref, s3_ref, np_ref=None):
    p1 = p1_ref[...]
    g1 = gt_ref[:, 0:64]
    g2 = gt_ref[:, 128:256]
    g3 = gt_ref[:, 256:512]
    s1 = g1 + b1_ref[...] - jnp.dot(p1, w1p_ref[...], precision=_HIGH)
    s1_ref[...] = s1
    s2 = (g2 + b2_ref[...]
          + jnp.dot(s1, w2x_ref[...], precision=_HIGH)
          - jnp.dot(p1, w2p_ref[...], precision=_HIGH))
    s2_ref[...] = s2
    s3 = (g3 + b3_ref[...]
          + jnp.dot(s2, w3x_ref[...], precision=_HIGH)
          - jnp.dot(p1, w3p_ref[...], precision=_HIGH))
    s3_ref[...] = s3
    if decode:
        h = jnp.maximum(jnp.dot(s3, wm_ref[...], precision=_HIGH)
                        + bm_ref[...], 0.0)
        motion = jnp.dot(h, wl_ref[...], precision=_HIGH) + bl_ref[...]
        np_ref[...] = p1 + motion


def _post(decode, gt, p1f, w1p, w2x, w2p, w3x, w3p,
          b1r, b2r, b3r, wm, bmr, wlp, blr):
    out_specs = [
        pl.BlockSpec((BM, 64), lambda i: (i, 0)),
        pl.BlockSpec((BM, 128), lambda i: (i, 0)),
        pl.BlockSpec((BM, 256), lambda i: (i, 0)),
    ]
    out_shape = [
        jax.ShapeDtypeStruct((M, 64), _f32),
        jax.ShapeDtypeStruct((M, 128), _f32),
        jax.ShapeDtypeStruct((M, 256), _f32),
    ]
    if decode:
        out_specs.append(pl.BlockSpec((BM, 8), lambda i: (i, 0)))
        out_shape.append(jax.ShapeDtypeStruct((M, 8), _f32))
    return pl.pallas_call(
        functools.partial(_post_body, decode),
        grid=(M // BM,),
        in_specs=[
            pl.BlockSpec((BM, CT), lambda i: (i, 0)),
            pl.BlockSpec((BM, 8), lambda i: (i, 0)),
            _full(8, 64), _full(64, 128), _full(8, 128),
            _full(128, 256), _full(8, 256),
            _full(1, 64), _full(1, 128), _full(1, 256),
            _full(256, 64), _full(1, 64), _full(64, 8), _full(1, 8),
        ],
        out_specs=out_specs,
        out_shape=out_shape,
    )(gt, p1f, w1p, w2x, w2p, w3x, w3p,
      b1r, b2r, b3r, wm, bmr, wlp, blr)


def kernel(frames, W1, b1, W2, b2, W3, b3, Wm, bm, Wl, bl):
    pad35 = ((0, 5), (0, 0))
    padc64 = ((0, 0), (0, 64))
    w1s = jnp.pad(W1[:64], padc64)
    w1p = jnp.pad(W1[64:], pad35)
    w1p_wide = jnp.pad(w1p, padc64)
    w2s, w2x, w2p = W2[:128], W2[128:192], jnp.pad(W2[192:], pad35)
    w3s, w3x, w3p = W3[:256], W3[256:384], jnp.pad(W3[384:], pad35)
    wlp = jnp.pad(Wl, ((0, 0), (0, 5)))
    blr = jnp.pad(bl, (0, 5))[None, :]
    b1r, b2r, b3r, bmr = b1[None, :], b2[None, :], b3[None, :], bm[None, :]

    framesP = jnp.pad(frames, ((0, 0), (0, 0), (0, 0), (0, 5)))
    framesT = framesP.transpose(0, 1, 3, 2)

    s1 = jnp.zeros((M, 64), _f32)
    s2 = jnp.zeros((M, 128), _f32)
    s3 = jnp.zeros((M, 256), _f32)

    preds = []
    p1n, p1t = framesP[:, 0], framesT[:, 0]
    p2t_t = p1t
    p1f = p1n.reshape(M, 8)
    p2f = p1f

    for t in range(T):
        idx_t = _knn(p1n, p2t_t)
        idx_flat = idx_t.transpose(0, 2, 1).reshape(M * K)
        at = _prep(s1, s2, s3, p2f,
                   w1s, w1p_wide, w2s, w2p, w3s, w3p)
        gt = _gather_max_kernel()(at, idx_flat)
        decode = t >= HALF
        outs = _post(decode, gt, p1f,
                     w1p, w2x, w2p, w3x, w3p,
                     b1r, b2r, b3r, Wm, bmr, wlp, blr)
        if decode:
            s1, s2, s3, newpf = outs
            preds.append(newpf[:, :3].reshape(B, N, 3))
        else:
            s1, s2, s3 = outs

        p2f = p1f
        p2t_t = p1t
        if t + 1 < HALF:
            p1n, p1t = framesP[:, t + 1], framesT[:, t + 1]
            p1f = p1n.reshape(M, 8)
        elif t + 1 == HALF:
            p1n, p1t = framesP[:, HALF - 1], framesT[:, HALF - 1]
            p1f = p1n.reshape(M, 8)
            p2t_t = p1t
            p2f = p1f
        else:
            p1f = newpf
            p1n = newpf.reshape(B, N, 8)
            p1t = p1n.transpose(0, 2, 1)

    return jnp.stack(preds, axis=1)


# speedup vs baseline: 21.7564x; 1.2932x over previous
import functools

import jax
import jax.numpy as jnp
from jax import lax
from jax.experimental import pallas as pl
from jax.experimental.pallas import tpu as pltpu
from jax.experimental.pallas import tpu_sc as plsc

B = 4
T = 8
N = 2048
K = 8
M = B * N
HALF = T // 2

NC = 2
NS = 16
NW = NC * NS
PTS_PER_W = M // NW
CHUNK = 8
NCH = PTS_PER_W // CHUNK

BN = 256
BM = 512

_f32 = jnp.float32
_HIGH = lax.Precision.HIGHEST


def _knn_body(p1_ref, p2t_ref, idx_ref):
    b = pl.program_id(0)
    p1 = p1_ref[0]
    p2t = p2t_ref[0]
    sq2 = jnp.sum(p2t * p2t, axis=0)
    dots = jnp.dot(p1, p2t, precision=_HIGH)
    scores = sq2[None, :] - 2.0 * dots
    u = lax.bitcast_convert_type(scores, jnp.int32)
    key = u ^ (lax.shift_right_arithmetic(u, 31) & jnp.int32(0x7FFFFFFF))
    cols = lax.broadcasted_iota(jnp.int32, (BN, N), 1)
    key = (key & jnp.int32(~0x7FF)) | cols
    base = b * N
    imax = jnp.int32(0x7FFFFFFF)
    for j in range(K):
        m = jnp.min(key, axis=1, keepdims=True)
        idx_ref[0, j, :] = (m[:, 0] & jnp.int32(0x7FF)) + base
        key = jnp.where(key == m, imax, key)


def _knn(p1n, p2t):
    return pl.pallas_call(
        _knn_body,
        grid=(B, N // BN),
        in_specs=[
            pl.BlockSpec((1, BN, 8), lambda b, nb: (b, nb, 0)),
            pl.BlockSpec((1, 8, N), lambda b, nb: (b, 0, 0)),
        ],
        out_specs=pl.BlockSpec((1, K, BN), lambda b, nb: (b, 0, nb)),
        out_shape=jax.ShapeDtypeStruct((B, K, N), jnp.int32),
    )(p1n, p2t)


CT = 512


def _prep_body(s1_ref, s2_ref, s3_ref, p2_ref,
               w1s_ref, w1p_ref, w2s_ref, w2p_ref, w3s_ref, w3p_ref,
               at_ref):
    p2 = p2_ref[...]
    at_ref[:, 0:128] = (jnp.dot(s1_ref[...], w1s_ref[...], precision=_HIGH)
                        + jnp.dot(p2, w1p_ref[...], precision=_HIGH))
    at_ref[:, 128:256] = (jnp.dot(s2_ref[...], w2s_ref[...], precision=_HIGH)
                          + jnp.dot(p2, w2p_ref[...], precision=_HIGH))
    at_ref[:, 256:512] = (jnp.dot(s3_ref[...], w3s_ref[...], precision=_HIGH)
                          + jnp.dot(p2, w3p_ref[...], precision=_HIGH))


def _full(r, c):
    return pl.BlockSpec((r, c), lambda i: (0, 0))


def _prep(s1, s2, s3, p2f, w1s, w1p, w2s, w2p, w3s, w3p):
    return pl.pallas_call(
        _prep_body,
        grid=(M // BM,),
        in_specs=[
            pl.BlockSpec((BM, 64), lambda i: (i, 0)),
            pl.BlockSpec((BM, 128), lambda i: (i, 0)),
            pl.BlockSpec((BM, 256), lambda i: (i, 0)),
            pl.BlockSpec((BM, 8), lambda i: (i, 0)),
            _full(64, 128), _full(8, 128),
            _full(128, 128), _full(8, 128),
            _full(256, 256), _full(8, 256),
        ],
        out_specs=pl.BlockSpec((BM, CT), lambda i: (i, 0)),
        out_shape=jax.ShapeDtypeStruct((M, CT), _f32),
    )(s1, s2, s3, p2f, w1s, w1p, w2s, w2p, w3s, w3p)


_LIVE_CH = tuple(range(4)) + tuple(range(8, CT // 16))


@functools.cache
def _gather_max_kernel():
    mesh = plsc.VectorSubcoreMesh(
        core_axis_name="c", subcore_axis_name="s",
        num_cores=NC, num_subcores=NS)

    @functools.partial(
        pl.kernel,
        out_type=jax.ShapeDtypeStruct((M, CT), _f32),
        mesh=mesh,
        scratch_types=[
            pltpu.VMEM((CHUNK * K,), jnp.int32),
            pltpu.VMEM((CHUNK * K,), jnp.int32),
            pltpu.VMEM((CHUNK * K, CT), _f32),
            pltpu.VMEM((CHUNK * K, CT), _f32),
            pltpu.VMEM((CHUNK, CT), _f32),
            pltpu.SemaphoreType.DMA,
            pltpu.SemaphoreType.DMA,
        ],
    )
    def _gather_max(at_hbm, idx_hbm, gt_hbm,
                    idx0, idx1, rows0, rows1, out_v, sem0, sem1):
        wid = lax.axis_index("s") * NC + lax.axis_index("c")
        base_pt = wid * PTS_PER_W

        def start(ci, ibuf, rbuf, sem):
            pt0 = base_pt + ci * CHUNK
            pltpu.sync_copy(idx_hbm.at[pl.ds(pt0 * K, CHUNK * K)], ibuf)
            pltpu.async_copy(at_hbm.at[ibuf], rbuf, sem)

        def finish(ci, ibuf, rbuf, sem):
            pltpu.make_async_copy(at_hbm.at[ibuf], rbuf, sem).wait()

            def pt_body(p, c2):
                row = p * K
                for ch in _LIVE_CH:
                    sl = pl.ds(ch * 16, 16)
                    v = rbuf[row, sl]
                    for j in range(1, K):
                        v = jnp.maximum(v, rbuf[row + j, sl])
                    out_v[p, sl] = v
                return c2

            lax.fori_loop(0, CHUNK, pt_body, 0)
            pt0 = base_pt + ci * CHUNK
            pltpu.sync_copy(out_v, gt_hbm.at[pl.ds(pt0, CHUNK)])

        start(0, idx0, rows0, sem0)
        start(1, idx1, rows1, sem1)

        def pair_body(i, carry):
            ci = 2 * i
            finish(ci, idx0, rows0, sem0)

            @pl.when(ci + 2 < NCH)
            def _():
                start(ci + 2, idx0, rows0, sem0)

            finish(ci + 1, idx1, rows1, sem1)

            @pl.when(ci + 3 < NCH)
            def _():
                start(ci + 3, idx1, rows1, sem1)

            return carry

        lax.fori_loop(0, NCH // 2, pair_body, 0)

    return _gather_max


def _post_body(decode, gt_ref, p1_ref,
               w1p_ref, w2x_ref, w2p_ref, w3x_ref, w3p_ref,
               b1_ref, b2_ref, b3_ref,
               w1s_ref, w1pw_ref, w2s_ref, w3s_ref,
               wm_ref, bm_ref, wl_ref, bl_ref,
               at_ref, np_ref=None):
    p1 = p1_ref[...]
    g1 = gt_ref[:, 0:64]
    g2 = gt_ref[:, 128:256]
    g3 = gt_ref[:, 256:512]
    s1 = g1 + b1_ref[...] - jnp.dot(p1, w1p_ref[...], precision=_HIGH)
    s2 = (g2 + b2_ref[...]
          + jnp.dot(s1, w2x_ref[...], precision=_HIGH)
          - jnp.dot(p1, w2p_ref[...], precision=_HIGH))
    s3 = (g3 + b3_ref[...]
          + jnp.dot(s2, w3x_ref[...], precision=_HIGH)
          - jnp.dot(p1, w3p_ref[...], precision=_HIGH))
    at_ref[:, 0:128] = (jnp.dot(s1, w1s_ref[...], precision=_HIGH)
                        + jnp.dot(p1, w1pw_ref[...], precision=_HIGH))
    at_ref[:, 128:256] = (jnp.dot(s2, w2s_ref[...], precision=_HIGH)
                          + jnp.dot(p1, w2p_ref[...], precision=_HIGH))
    at_ref[:, 256:512] = (jnp.dot(s3, w3s_ref[...], precision=_HIGH)
                          + jnp.dot(p1, w3p_ref[...], precision=_HIGH))
    if decode:
        h = jnp.maximum(jnp.dot(s3, wm_ref[...], precision=_HIGH)
                        + bm_ref[...], 0.0)
        motion = jnp.dot(h, wl_ref[...], precision=_HIGH) + bl_ref[...]
        np_ref[...] = p1 + motion


def _post(decode, gt, p1f, w1p, w2x, w2p, w3x, w3p,
          b1r, b2r, b3r, w1s, w1pw, w2s, w3s, wm, bmr, wlp, blr):
    out_specs = [pl.BlockSpec((BM, CT), lambda i: (i, 0))]
    out_shape = [jax.ShapeDtypeStruct((M, CT), _f32)]
    if decode:
        out_specs.append(pl.BlockSpec((BM, 8), lambda i: (i, 0)))
        out_shape.append(jax.ShapeDtypeStruct((M, 8), _f32))
    return pl.pallas_call(
        functools.partial(_post_body, decode),
        grid=(M // BM,),
        in_specs=[
            pl.BlockSpec((BM, CT), lambda i: (i, 0)),
            pl.BlockSpec((BM, 8), lambda i: (i, 0)),
            _full(8, 64), _full(64, 128), _full(8, 128),
            _full(128, 256), _full(8, 256),
            _full(1, 64), _full(1, 128), _full(1, 256),
            _full(64, 128), _full(8, 128), _full(128, 128), _full(256, 256),
            _full(256, 64), _full(1, 64), _full(64, 8), _full(1, 8),
        ],
        out_specs=out_specs,
        out_shape=out_shape,
    )(gt, p1f, w1p, w2x, w2p, w3x, w3p,
      b1r, b2r, b3r, w1s, w1pw, w2s, w3s, wm, bmr, wlp, blr)


def kernel(frames, W1, b1, W2, b2, W3, b3, Wm, bm, Wl, bl):
    pad35 = ((0, 5), (0, 0))
    padc64 = ((0, 0), (0, 64))
    w1s = jnp.pad(W1[:64], padc64)
    w1p = jnp.pad(W1[64:], pad35)
    w1p_wide = jnp.pad(w1p, padc64)
    w2s, w2x, w2p = W2[:128], W2[128:192], jnp.pad(W2[192:], pad35)
    w3s, w3x, w3p = W3[:256], W3[256:384], jnp.pad(W3[384:], pad35)
    wlp = jnp.pad(Wl, ((0, 0), (0, 5)))
    blr = jnp.pad(bl, (0, 5))[None, :]
    b1r, b2r, b3r, bmr = b1[None, :], b2[None, :], b3[None, :], bm[None, :]

    framesP = jnp.pad(frames, ((0, 0), (0, 0), (0, 0), (0, 5)))
    framesT = framesP.transpose(0, 1, 3, 2)

    preds = []
    p1n, p1t = framesP[:, 0], framesT[:, 0]
    p2t_t = p1t
    p1f = p1n.reshape(M, 8)

    z1 = jnp.zeros((M, 64), _f32)
    z2 = jnp.zeros((M, 128), _f32)
    z3 = jnp.zeros((M, 256), _f32)
    at = _prep(z1, z2, z3, p1f, w1s, w1p_wide, w2s, w2p, w3s, w3p)

    for t in range(T):
        idx_t = _knn(p1n, p2t_t)
        idx_flat = idx_t.transpose(0, 2, 1).reshape(M * K)
        gt = _gather_max_kernel()(at, idx_flat)
        decode = t >= HALF
        outs = _post(decode, gt, p1f,
                     w1p, w2x, w2p, w3x, w3p,
                     b1r, b2r, b3r, w1s, w1p_wide, w2s, w3s,
                     Wm, bmr, wlp, blr)
        if decode:
            at, newpf = outs
            preds.append(newpf[:, :3].reshape(B, N, 3))
        else:
            at = outs[0]

        p2t_t = p1t
        if t + 1 < HALF:
            p1n, p1t = framesP[:, t + 1], framesT[:, t + 1]
            p1f = p1n.reshape(M, 8)
        elif t + 1 == HALF:
            p1n, p1t = framesP[:, HALF - 1], framesT[:, HALF - 1]
            p1f = p1n.reshape(M, 8)
            p2t_t = p1t
        else:
            p1f = newpf
            p1n = newpf.reshape(B, N, 8)
            p1t = p1n.transpose(0, 2, 1)

    return jnp.stack(preds, axis=1)


# speedup vs baseline: 24.0128x; 1.1037x over previous
import functools

import jax
import jax.numpy as jnp
from jax import lax
from jax.experimental import pallas as pl
from jax.experimental.pallas import tpu as pltpu
from jax.experimental.pallas import tpu_sc as plsc

B = 4
T = 8
N = 2048
K = 8
M = B * N
HALF = T // 2

NC = 2
NS = 16
NW = NC * NS
PTS_PER_W = M // NW
CHUNK = 8
NCH = PTS_PER_W // CHUNK

BN = 512
BM = 1024

_f32 = jnp.float32
_HIGH = lax.Precision.HIGHEST


def _knn_body(p1_ref, p2t_ref, idx_ref):
    b = pl.program_id(0)
    p1 = p1_ref[0]
    p2t = p2t_ref[0]
    sq2 = jnp.sum(p2t * p2t, axis=0)
    dots = jnp.dot(p1, p2t, precision=_HIGH)
    scores = sq2[None, :] - 2.0 * dots
    u = lax.bitcast_convert_type(scores, jnp.int32)
    key = u ^ (lax.shift_right_arithmetic(u, 31) & jnp.int32(0x7FFFFFFF))
    cols = lax.broadcasted_iota(jnp.int32, (BN, N), 1)
    key = (key & jnp.int32(~0x7FF)) | cols
    base = b * N
    imax = jnp.int32(0x7FFFFFFF)
    for j in range(K):
        m = jnp.min(key, axis=1, keepdims=True)
        idx_ref[0, j, :] = (m[:, 0] & jnp.int32(0x7FF)) + base
        key = jnp.where(key == m, imax, key)


def _knn(p1n, p2t):
    return pl.pallas_call(
        _knn_body,
        grid=(B, N // BN),
        in_specs=[
            pl.BlockSpec((1, BN, 8), lambda b, nb: (b, nb, 0)),
            pl.BlockSpec((1, 8, N), lambda b, nb: (b, 0, 0)),
        ],
        out_specs=pl.BlockSpec((1, K, BN), lambda b, nb: (b, 0, nb)),
        out_shape=jax.ShapeDtypeStruct((B, K, N), jnp.int32),
    )(p1n, p2t)


CT = 512


def _prep_body(s1_ref, s2_ref, s3_ref, p2_ref,
               w1s_ref, w1p_ref, w2s_ref, w2p_ref, w3s_ref, w3p_ref,
               at_ref):
    p2 = p2_ref[...]
    at_ref[:, 0:128] = (jnp.dot(s1_ref[...], w1s_ref[...], precision=_HIGH)
                        + jnp.dot(p2, w1p_ref[...], precision=_HIGH))
    at_ref[:, 128:256] = (jnp.dot(s2_ref[...], w2s_ref[...], precision=_HIGH)
                          + jnp.dot(p2, w2p_ref[...], precision=_HIGH))
    at_ref[:, 256:512] = (jnp.dot(s3_ref[...], w3s_ref[...], precision=_HIGH)
                          + jnp.dot(p2, w3p_ref[...], precision=_HIGH))


def _full(r, c):
    return pl.BlockSpec((r, c), lambda i: (0, 0))


def _prep(s1, s2, s3, p2f, w1s, w1p, w2s, w2p, w3s, w3p):
    return pl.pallas_call(
        _prep_body,
        grid=(M // BM,),
        in_specs=[
            pl.BlockSpec((BM, 64), lambda i: (i, 0)),
            pl.BlockSpec((BM, 128), lambda i: (i, 0)),
            pl.BlockSpec((BM, 256), lambda i: (i, 0)),
            pl.BlockSpec((BM, 8), lambda i: (i, 0)),
            _full(64, 128), _full(8, 128),
            _full(128, 128), _full(8, 128),
            _full(256, 256), _full(8, 256),
        ],
        out_specs=pl.BlockSpec((BM, CT), lambda i: (i, 0)),
        out_shape=jax.ShapeDtypeStruct((M, CT), _f32),
    )(s1, s2, s3, p2f, w1s, w1p, w2s, w2p, w3s, w3p)


_LIVE_CH = tuple(range(4)) + tuple(range(8, CT // 16))


@functools.cache
def _gather_max_kernel():
    mesh = plsc.VectorSubcoreMesh(
        core_axis_name="c", subcore_axis_name="s",
        num_cores=NC, num_subcores=NS)

    @functools.partial(
        pl.kernel,
        out_type=jax.ShapeDtypeStruct((M, CT), _f32),
        mesh=mesh,
        scratch_types=[
            pltpu.VMEM((CHUNK * K,), jnp.int32),
            pltpu.VMEM((CHUNK * K,), jnp.int32),
            pltpu.VMEM((CHUNK * K, CT), _f32),
            pltpu.VMEM((CHUNK * K, CT), _f32),
            pltpu.VMEM((CHUNK, CT), _f32),
            pltpu.SemaphoreType.DMA,
            pltpu.SemaphoreType.DMA,
        ],
    )
    def _gather_max(at_hbm, idx_hbm, gt_hbm,
                    idx0, idx1, rows0, rows1, out_v, sem0, sem1):
        wid = lax.axis_index("s") * NC + lax.axis_index("c")
        base_pt = wid * PTS_PER_W

        def start(ci, ibuf, rbuf, sem):
            pt0 = base_pt + ci * CHUNK
            pltpu.sync_copy(idx_hbm.at[pl.ds(pt0 * K, CHUNK * K)], ibuf)
            pltpu.async_copy(at_hbm.at[ibuf], rbuf, sem)

        def finish(ci, ibuf, rbuf, sem):
            pltpu.make_async_copy(at_hbm.at[ibuf], rbuf, sem).wait()

            def pt_body(p, c2):
                row = p * K
                for ch in _LIVE_CH:
                    sl = pl.ds(ch * 16, 16)
                    v = rbuf[row, sl]
                    for j in range(1, K):
                        v = jnp.maximum(v, rbuf[row + j, sl])
                    out_v[p, sl] = v
                return c2

            lax.fori_loop(0, CHUNK, pt_body, 0)
            pt0 = base_pt + ci * CHUNK
            pltpu.sync_copy(out_v, gt_hbm.at[pl.ds(pt0, CHUNK)])

        start(0, idx0, rows0, sem0)
        start(1, idx1, rows1, sem1)

        def pair_body(i, carry):
            ci = 2 * i
            finish(ci, idx0, rows0, sem0)

            @pl.when(ci + 2 < NCH)
            def _():
                start(ci + 2, idx0, rows0, sem0)

            finish(ci + 1, idx1, rows1, sem1)

            @pl.when(ci + 3 < NCH)
            def _():
                start(ci + 3, idx1, rows1, sem1)

            return carry

        lax.fori_loop(0, NCH // 2, pair_body, 0)

    return _gather_max


def _post_body(decode, gt_ref, p1_ref,
               w1p_ref, w2x_ref, w2p_ref, w3x_ref, w3p_ref,
               b1_ref, b2_ref, b3_ref,
               w1s_ref, w1pw_ref, w2s_ref, w3s_ref,
               wm_ref, bm_ref, wl_ref, bl_ref,
               at_ref, np_ref=None):
    p1 = p1_ref[...]
    g1 = gt_ref[:, 0:64]
    g2 = gt_ref[:, 128:256]
    g3 = gt_ref[:, 256:512]
    s1 = g1 + b1_ref[...] - jnp.dot(p1, w1p_ref[...], precision=_HIGH)
    s2 = (g2 + b2_ref[...]
          + jnp.dot(s1, w2x_ref[...], precision=_HIGH)
          - jnp.dot(p1, w2p_ref[...], precision=_HIGH))
    s3 = (g3 + b3_ref[...]
          + jnp.dot(s2, w3x_ref[...], precision=_HIGH)
          - jnp.dot(p1, w3p_ref[...], precision=_HIGH))
    at_ref[:, 0:128] = (jnp.dot(s1, w1s_ref[...], precision=_HIGH)
                        + jnp.dot(p1, w1pw_ref[...], precision=_HIGH))
    at_ref[:, 128:256] = (jnp.dot(s2, w2s_ref[...], precision=_HIGH)
                          + jnp.dot(p1, w2p_ref[...], precision=_HIGH))
    at_ref[:, 256:512] = (jnp.dot(s3, w3s_ref[...], precision=_HIGH)
                          + jnp.dot(p1, w3p_ref[...], precision=_HIGH))
    if decode:
        h = jnp.maximum(jnp.dot(s3, wm_ref[...], precision=_HIGH)
                        + bm_ref[...], 0.0)
        motion = jnp.dot(h, wl_ref[...], precision=_HIGH) + bl_ref[...]
        np_ref[...] = p1 + motion


def _post(decode, gt, p1f, w1p, w2x, w2p, w3x, w3p,
          b1r, b2r, b3r, w1s, w1pw, w2s, w3s, wm, bmr, wlp, blr):
    out_specs = [pl.BlockSpec((BM, CT), lambda i: (i, 0))]
    out_shape = [jax.ShapeDtypeStruct((M, CT), _f32)]
    if decode:
        out_specs.append(pl.BlockSpec((BM, 8), lambda i: (i, 0)))
        out_shape.append(jax.ShapeDtypeStruct((M, 8), _f32))
    return pl.pallas_call(
        functools.partial(_post_body, decode),
        grid=(M // BM,),
        in_specs=[
            pl.BlockSpec((BM, CT), lambda i: (i, 0)),
            pl.BlockSpec((BM, 8), lambda i: (i, 0)),
            _full(8, 64), _full(64, 128), _full(8, 128),
            _full(128, 256), _full(8, 256),
            _full(1, 64), _full(1, 128), _full(1, 256),
            _full(64, 128), _full(8, 128), _full(128, 128), _full(256, 256),
            _full(256, 64), _full(1, 64), _full(64, 8), _full(1, 8),
        ],
        out_specs=out_specs,
        out_shape=out_shape,
    )(gt, p1f, w1p, w2x, w2p, w3x, w3p,
      b1r, b2r, b3r, w1s, w1pw, w2s, w3s, wm, bmr, wlp, blr)


def kernel(frames, W1, b1, W2, b2, W3, b3, Wm, bm, Wl, bl):
    pad35 = ((0, 5), (0, 0))
    padc64 = ((0, 0), (0, 64))
    w1s = jnp.pad(W1[:64], padc64)
    w1p = jnp.pad(W1[64:], pad35)
    w1p_wide = jnp.pad(w1p, padc64)
    w2s, w2x, w2p = W2[:128], W2[128:192], jnp.pad(W2[192:], pad35)
    w3s, w3x, w3p = W3[:256], W3[256:384], jnp.pad(W3[384:], pad35)
    wlp = jnp.pad(Wl, ((0, 0), (0, 5)))
    blr = jnp.pad(bl, (0, 5))[None, :]
    b1r, b2r, b3r, bmr = b1[None, :], b2[None, :], b3[None, :], bm[None, :]

    framesP = jnp.pad(frames, ((0, 0), (0, 0), (0, 0), (0, 5)))
    framesT = framesP.transpose(0, 1, 3, 2)

    preds = []
    p1n, p1t = framesP[:, 0], framesT[:, 0]
    p2t_t = p1t
    p1f = p1n.reshape(M, 8)

    z1 = jnp.zeros((M, 64), _f32)
    z2 = jnp.zeros((M, 128), _f32)
    z3 = jnp.zeros((M, 256), _f32)
    at = _prep(z1, z2, z3, p1f, w1s, w1p_wide, w2s, w2p, w3s, w3p)

    enc_idx = []
    for t in range(HALF):
        p1_enc = framesP[:, t]
        p2t_enc = framesT[:, max(t - 1, 0)]
        enc_idx.append(
            _knn(p1_enc, p2t_enc).transpose(0, 2, 1).reshape(M * K))

    for t in range(T):
        if t < HALF:
            idx_flat = enc_idx[t]
        else:
            idx_t = _knn(p1n, p2t_t)
            idx_flat = idx_t.transpose(0, 2, 1).reshape(M * K)
        gt = _gather_max_kernel()(at, idx_flat)
        decode = t >= HALF
        outs = _post(decode, gt, p1f,
                     w1p, w2x, w2p, w3x, w3p,
                     b1r, b2r, b3r, w1s, w1p_wide, w2s, w3s,
                     Wm, bmr, wlp, blr)
        if decode:
            at, newpf = outs
            preds.append(newpf[:, :3].reshape(B, N, 3))
        else:
            at = outs[0]

        p2t_t = p1t
        if t + 1 < HALF:
            p1n, p1t = framesP[:, t + 1], framesT[:, t + 1]
            p1f = p1n.reshape(M, 8)
        elif t + 1 == HALF:
            p1n, p1t = framesP[:, HALF - 1], framesT[:, HALF - 1]
            p1f = p1n.reshape(M, 8)
            p2t_t = p1t
        else:
            p1f = newpf
            p1n = newpf.reshape(B, N, 8)
            p1t = p1n.transpose(0, 2, 1)

    return jnp.stack(preds, axis=1)
